# baseline (speedup 1.0000x reference)
import functools

import jax
import jax.numpy as jnp
from jax import lax
from jax.experimental import pallas as pl
from jax.experimental.pallas import tpu as pltpu
from jax.experimental.pallas import tpu_sc as plsc

_NUM_ITEMS = 1000000
_D = 128
_B = 16384
_H = 64

_NC = 2
_NS = 16
_NW = _NC * _NS
_HALF = _B // 2
_BPW = _HALF // _NW
_CHUNK = 128
_NCHUNK = _BPW // _CHUNK


def _sc_gather_half(table, idx_grouped):
    mesh = plsc.VectorSubcoreMesh(core_axis_name="c", subcore_axis_name="s")

    @functools.partial(
        pl.kernel,
        mesh=mesh,
        out_type=jax.ShapeDtypeStruct((_HALF, _D), jnp.float32),
        scratch_types=[
            pltpu.VMEM((_NCHUNK, _CHUNK), jnp.int32),
            pltpu.VMEM((_BPW, _D), jnp.float32),
        ]
        + [pltpu.SemaphoreType.DMA] * _NCHUNK
        + [pltpu.SemaphoreType.DMA],
    )
    def k(table_hbm, idx_hbm, out_hbm, idx_v, rows_v, *sems):
        gsems, wsem = sems[:_NCHUNK], sems[_NCHUNK]
        wid = lax.axis_index("s") * _NC + lax.axis_index("c")
        pltpu.sync_copy(idx_hbm.at[wid], idx_v)
        gathers = []
        for j in range(_NCHUNK):
            gathers.append(
                pltpu.async_copy(
                    table_hbm.at[idx_v.at[j]],
                    rows_v.at[pl.ds(j * _CHUNK, _CHUNK)],
                    gsems[j],
                )
            )
        writes = []
        for j in range(_NCHUNK):
            gathers[j].wait()
            writes.append(
                pltpu.async_copy(
                    rows_v.at[pl.ds(j * _CHUNK, _CHUNK)],
                    out_hbm.at[pl.ds(wid * _BPW + j * _CHUNK, _CHUNK)],
                    wsem,
                )
            )
        for c in writes:
            c.wait()

    return k(table, idx_grouped)


_BLK = 4096
_HGRID = _HALF // _BLK


def _tc_body(t_ref, w1_ref, b1_ref, w2_ref, b2_ref, g_ref, o_ref):
    trow = t_ref[0]
    ht = jnp.maximum(w1_ref[...] * trow + b1_ref[...], 0.0)
    s = jnp.tanh(
        lax.dot_general(
            ht, w2_ref[...], (((0,), (0,)), ((), ())),
            preferred_element_type=jnp.float32,
        )
        + b2_ref[...]
    )
    o_ref[...] = g_ref[...] + s


_COMMON_SPECS = [
    pl.BlockSpec((_H, 1), lambda i: (0, 0)),
    pl.BlockSpec((_H, 1), lambda i: (0, 0)),
    pl.BlockSpec((_H, _D), lambda i: (0, 0)),
    pl.BlockSpec((1, _D), lambda i: (0, 0)),
]


def _tc_mlp_add_half0(ts3d, W1t, b1t, W2, b2, g_half):
    def body(t_ref, w1_ref, b1_ref, w2_ref, b2_ref, g_ref, o_ref):
        _tc_body(t_ref, w1_ref, b1_ref, w2_ref, b2_ref, g_ref, o_ref)

    return pl.pallas_call(
        body,
        grid=(_HGRID,),
        in_specs=[pl.BlockSpec((1, 1, _BLK), lambda i: (i, 0, 0))]
        + _COMMON_SPECS
        + [pl.BlockSpec((_BLK, _D), lambda i: (i, 0))],
        out_specs=pl.BlockSpec((_BLK, _D), lambda i: (i, 0)),
        out_shape=jax.ShapeDtypeStruct((_B, _D), jnp.float32),
        compiler_params=pltpu.CompilerParams(
            dimension_semantics=("parallel",),
        ),
    )(ts3d, W1t, b1t, W2, b2, g_half)


def _tc_mlp_add_half1(ts3d, W1t, b1t, W2, b2, g_half, acc):
    def body(t_ref, w1_ref, b1_ref, w2_ref, b2_ref, g_ref, a_ref, o_ref):
        del a_ref
        _tc_body(t_ref, w1_ref, b1_ref, w2_ref, b2_ref, g_ref, o_ref)

    return pl.pallas_call(
        body,
        grid=(_HGRID,),
        in_specs=[pl.BlockSpec((1, 1, _BLK), lambda i: (i, 0, 0))]
        + _COMMON_SPECS
        + [
            pl.BlockSpec((_BLK, _D), lambda i: (i, 0)),
            pl.BlockSpec(memory_space=pl.ANY),
        ],
        out_specs=pl.BlockSpec((_BLK, _D), lambda i: (i + _HGRID, 0)),
        out_shape=jax.ShapeDtypeStruct((_B, _D), jnp.float32),
        input_output_aliases={6: 0},
        compiler_params=pltpu.CompilerParams(
            dimension_semantics=("parallel",),
        ),
    )(ts3d, W1t, b1t, W2, b2, g_half, acc)


def kernel(indices, timestamps, base_embeddings, W1, b1, W2, b2):
    idx32 = indices.astype(jnp.int32)
    idx_a = idx32[:_HALF].reshape(_NW, _NCHUNK, _CHUNK)
    idx_b = idx32[_HALF:].reshape(_NW, _NCHUNK, _CHUNK)
    g_a = _sc_gather_half(base_embeddings, idx_a)
    g_b = _sc_gather_half(base_embeddings, idx_b)
    ts = timestamps.astype(jnp.float32)
    ts_a = ts[:_HALF].reshape(_HGRID, 1, _BLK)
    ts_b = ts[_HALF:].reshape(_HGRID, 1, _BLK)
    w1t = W1.reshape(_H, 1)
    b1t = b1.reshape(_H, 1)
    b2r = b2.reshape(1, _D)
    out_a = _tc_mlp_add_half0(ts_a, w1t, b1t, W2, b2r, g_a)
    return _tc_mlp_add_half1(ts_b, w1t, b1t, W2, b2r, g_b, out_a)


# speedup vs baseline: 1.1101x; 1.1101x over previous
import functools

import jax
import jax.numpy as jnp
from jax import lax
from jax.experimental import pallas as pl
from jax.experimental.pallas import tpu as pltpu
from jax.experimental.pallas import tpu_sc as plsc

_NUM_ITEMS = 1000000
_D = 128
_B = 16384
_H = 64

_NC = 2
_NS = 16
_NW = _NC * _NS
_BPW = _B // _NW
_CHUNK = 128
_NCHUNK = _BPW // _CHUNK


def _sc_gather(table, idx):
    mesh = plsc.VectorSubcoreMesh(core_axis_name="c", subcore_axis_name="s")

    @functools.partial(
        pl.kernel,
        mesh=mesh,
        out_type=jax.ShapeDtypeStruct((_B, _D), jnp.float32),
        scratch_types=[
            pltpu.VMEM((_BPW,), jnp.int32),
            pltpu.VMEM((_BPW, _D), jnp.float32),
        ]
        + [pltpu.SemaphoreType.DMA] * _NCHUNK
        + [pltpu.SemaphoreType.DMA],
    )
    def k(table_hbm, idx_hbm, out_hbm, idx_v, rows_v, *sems):
        gsems, wsem = sems[:_NCHUNK], sems[_NCHUNK]
        wid = lax.axis_index("s") * _NC + lax.axis_index("c")
        base = wid * _BPW
        pltpu.sync_copy(idx_hbm.at[pl.ds(base, _BPW)], idx_v)
        gathers = []
        for j in range(_NCHUNK):
            gathers.append(
                pltpu.async_copy(
                    table_hbm.at[idx_v.at[pl.ds(j * _CHUNK, _CHUNK)]],
                    rows_v.at[pl.ds(j * _CHUNK, _CHUNK)],
                    gsems[j],
                )
            )
        writes = []
        for j in range(_NCHUNK):
            gathers[j].wait()
            writes.append(
                pltpu.async_copy(
                    rows_v.at[pl.ds(j * _CHUNK, _CHUNK)],
                    out_hbm.at[pl.ds(base + j * _CHUNK, _CHUNK)],
                    wsem,
                )
            )
        for c in writes:
            c.wait()

    return k(table, idx)


_BLK = 8192
_GRID = _B // _BLK


def _tc_mlp_add(ts, W1t, b1t, W2, b2, gathered):
    def body(t_ref, w1_ref, b1_ref, w2_ref, b2_ref, g_ref, o_ref):
        trow = t_ref[...].reshape(1, _BLK)
        ht = jnp.maximum(w1_ref[...] * trow + b1_ref[...], 0.0)
        s = jnp.tanh(
            lax.dot_general(
                ht, w2_ref[...], (((0,), (0,)), ((), ())),
                preferred_element_type=jnp.float32,
            )
            + b2_ref[...]
        )
        o_ref[...] = g_ref[...] + s

    return pl.pallas_call(
        body,
        grid=(_GRID,),
        in_specs=[
            pl.BlockSpec((_BLK,), lambda i: (i,)),
            pl.BlockSpec((_H, 1), lambda i: (0, 0)),
            pl.BlockSpec((_H, 1), lambda i: (0, 0)),
            pl.BlockSpec((_H, _D), lambda i: (0, 0)),
            pl.BlockSpec((1, _D), lambda i: (0, 0)),
            pl.BlockSpec((_BLK, _D), lambda i: (i, 0)),
        ],
        out_specs=pl.BlockSpec((_BLK, _D), lambda i: (i, 0)),
        out_shape=jax.ShapeDtypeStruct((_B, _D), jnp.float32),
        compiler_params=pltpu.CompilerParams(
            dimension_semantics=("parallel",),
        ),
    )(ts, W1t, b1t, W2, b2, gathered)


def kernel(indices, timestamps, base_embeddings, W1, b1, W2, b2):
    idx32 = indices.astype(jnp.int32)
    gathered = _sc_gather(base_embeddings, idx32)
    ts = timestamps.astype(jnp.float32)
    return _tc_mlp_add(
        ts,
        W1.reshape(_H, 1),
        b1.reshape(_H, 1),
        W2,
        b2.reshape(1, _D),
        gathered,
    )
